# baseline (speedup 1.0000x reference)
import functools

import jax
import jax.numpy as jnp
import numpy as np
from jax import lax
from jax.experimental import pallas as pl
from jax.experimental.pallas import tpu as pltpu
from jax.experimental.pallas import tpu_sc as plsc

N = 10000
E = 320000
H = 128
NF = 128
SHIFT = float(np.log(2.0))

NC = 2
NS = 16
NW = NC * NS
PER_W = E // NW
CHUNK = 80
NCH = PER_W // CHUNK
SUB = 2
LCHUNK = CHUNK * SUB
NFULL = 62
NPAD = 10240
ROWS_PER_TILE = NPAD // NS


def _sc_segment_sum(e, dst3, zeros):
    mesh = plsc.VectorSubcoreMesh(core_axis_name="c", subcore_axis_name="s")

    @functools.partial(
        pl.kernel,
        out_type=jax.ShapeDtypeStruct((NC, NPAD, H), jnp.float32),
        mesh=mesh,
        scratch_types=[
            pltpu.VMEM_SHARED((NPAD, H), jnp.float32),
            pltpu.VMEM((2, LCHUNK, H), jnp.float32),
            pltpu.VMEM((2, SUB, CHUNK), jnp.int32),
            pltpu.SemaphoreType.DMA((2,)),
            pltpu.SemaphoreType.DMA((2,)),
            pltpu.SemaphoreType.DMA((2,)),
        ],
    )
    def seg(e_hbm, dst_hbm, zero_hbm, out_hbm, acc, rows, idx, lsem, isem, ssem):
        cid = lax.axis_index("c")
        sid = lax.axis_index("s")
        w = cid * NS + sid
        ebase = w * PER_W

        def row_copy(j, slot):
            return pltpu.make_async_copy(
                e_hbm.at[pl.ds(ebase + j * LCHUNK, LCHUNK)], rows.at[slot],
                lsem.at[slot],
            )

        def idx_copy(j, slot):
            return pltpu.make_async_copy(
                dst_hbm.at[w, j], idx.at[slot], isem.at[slot]
            )

        def scat_copy(k, slot):
            return pltpu.make_async_copy(
                rows.at[slot, pl.ds(k * CHUNK, CHUNK)],
                acc.at[idx.at[slot, k]], ssem.at[slot],
            )

        row_copy(0, 0).start()
        idx_copy(0, 0).start()
        pltpu.sync_copy(zero_hbm, acc.at[pl.ds(sid * ROWS_PER_TILE, ROWS_PER_TILE)])
        plsc.subcore_barrier()

        def body(j, carry):
            slot = lax.rem(j, 2)
            row_copy(j, slot).wait()
            idx_copy(j, slot).wait()
            for k in range(SUB):
                scat_copy(k, slot).start(add=True)

            @pl.when(j >= 1)
            def _():
                for k in range(SUB):
                    scat_copy(k, 1 - slot).wait()

            @pl.when(j + 1 < NFULL)
            def _():
                row_copy(j + 1, 1 - slot).start()
                idx_copy(j + 1, 1 - slot).start()

            return carry

        lax.fori_loop(0, NFULL, body, 0)

        def tail_row_copy():
            return pltpu.make_async_copy(
                e_hbm.at[pl.ds(ebase + NFULL * LCHUNK, CHUNK)],
                rows.at[0, pl.ds(0, CHUNK)], lsem.at[0],
            )

        tail_row_copy().start()
        idx_copy(NFULL, 0).start()
        for k in range(SUB):
            scat_copy(k, 1).wait()
        tail_row_copy().wait()
        idx_copy(NFULL, 0).wait()
        scat_copy(0, 0).start(add=True)
        scat_copy(0, 0).wait()
        plsc.subcore_barrier()

        r0 = sid * ROWS_PER_TILE
        pltpu.sync_copy(
            acc.at[pl.ds(r0, ROWS_PER_TILE)], out_hbm.at[cid, pl.ds(r0, ROWS_PER_TILE)]
        )

    return seg(e, dst3, zeros)


BLK = 1000


def _tc_body(p_ref, v_ref, w1_ref, b1_ref, w2_ref, b2_ref, lnw_ref, lnb_ref, out_ref):
    x = p_ref[0] + p_ref[1]
    h = jnp.dot(x, w1_ref[...], preferred_element_type=jnp.float32,
                precision=lax.Precision.HIGHEST) + b1_ref[...]
    s = jnp.maximum(h, 0.0) + jnp.log1p(jnp.exp(-jnp.abs(h))) - SHIFT
    y = jnp.dot(s, w2_ref[...], preferred_element_type=jnp.float32,
                precision=lax.Precision.HIGHEST) + b2_ref[...]
    mu = jnp.mean(y, axis=-1, keepdims=True)
    yc = y - mu
    var = jnp.mean(yc * yc, axis=-1, keepdims=True)
    out_ref[...] = v_ref[...] + yc * lax.rsqrt(var + 1e-5) * lnw_ref[...] + lnb_ref[...]


def _tc_mlp(partials, v, W1, b1, W2, b2, lnw, lnb):
    return pl.pallas_call(
        _tc_body,
        grid=(N // BLK,),
        in_specs=[
            pl.BlockSpec((NC, BLK, H), lambda i: (0, i, 0)),
            pl.BlockSpec((BLK, H), lambda i: (i, 0)),
            pl.BlockSpec((H, H), lambda i: (0, 0)),
            pl.BlockSpec((1, H), lambda i: (0, 0)),
            pl.BlockSpec((H, H), lambda i: (0, 0)),
            pl.BlockSpec((1, H), lambda i: (0, 0)),
            pl.BlockSpec((1, H), lambda i: (0, 0)),
            pl.BlockSpec((1, H), lambda i: (0, 0)),
        ],
        out_specs=pl.BlockSpec((BLK, H), lambda i: (i, 0)),
        out_shape=jax.ShapeDtypeStruct((N, H), jnp.float32),
    )(partials, v, W1, b1, W2, b2, lnw, lnb)


def kernel(v, e, edge_index, v1_size, W1_1, b1_1, W1_2, b1_2, ln_w, ln_b):
    del v1_size
    dst4 = jnp.pad(
        edge_index[1].reshape(NW, PER_W), ((0, 0), (0, LCHUNK - CHUNK))
    ).reshape(NW, NFULL + 1, SUB, CHUNK)
    zeros = jnp.zeros((ROWS_PER_TILE, H), jnp.float32)
    partials = _sc_segment_sum(e, dst4, zeros)
    return _tc_mlp(
        partials, v, W1_1, b1_1.reshape(1, H), W1_2, b1_2.reshape(1, H),
        ln_w.reshape(1, H), ln_b.reshape(1, H),
    )


# speedup vs baseline: 1.1447x; 1.1447x over previous
import functools

import jax
import jax.numpy as jnp
import numpy as np
from jax import lax
from jax.experimental import pallas as pl
from jax.experimental.pallas import tpu as pltpu
from jax.experimental.pallas import tpu_sc as plsc

N = 10000
E = 320000
H = 128
NF = 128
SHIFT = float(np.log(2.0))

NC = 2
NS = 16
NW = NC * NS
PER_W = E // NW
CHUNK = 80
NCH = PER_W // CHUNK
SUB = 2
LCHUNK = CHUNK * SUB
NFULL = 62
NPAD = 10240
ROWS_PER_TILE = NPAD // NS


def _sc_segment_sum(e, dst3, zeros):
    mesh = plsc.VectorSubcoreMesh(core_axis_name="c", subcore_axis_name="s")

    @functools.partial(
        pl.kernel,
        out_type=jax.ShapeDtypeStruct((NC, NPAD, H), jnp.float32),
        mesh=mesh,
        scratch_types=[
            pltpu.VMEM_SHARED((NPAD, H), jnp.float32),
            pltpu.VMEM((2, LCHUNK, H), jnp.float32),
            pltpu.VMEM((2, SUB, CHUNK), jnp.int32),
            pltpu.SemaphoreType.DMA((2,)),
            pltpu.SemaphoreType.DMA((2,)),
            pltpu.SemaphoreType.DMA((2,)),
        ],
    )
    def seg(e_hbm, dst_hbm, zero_hbm, out_hbm, acc, rows, idx, lsem, isem, ssem):
        cid = lax.axis_index("c")
        sid = lax.axis_index("s")
        w = cid * NS + sid
        ebase = w * PER_W

        def row_copy(j, slot):
            return pltpu.make_async_copy(
                e_hbm.at[pl.ds(ebase + j * LCHUNK, LCHUNK)], rows.at[slot],
                lsem.at[slot],
            )

        def idx_copy(j, slot):
            return pltpu.make_async_copy(
                dst_hbm.at[w, j], idx.at[slot], isem.at[slot]
            )

        def scat_copy(k, slot):
            return pltpu.make_async_copy(
                rows.at[slot, pl.ds(k * CHUNK, CHUNK)],
                acc.at[idx.at[slot, k]], ssem.at[slot],
            )

        row_copy(0, 0).start()
        idx_copy(0, 0).start()
        pltpu.sync_copy(zero_hbm, acc.at[pl.ds(sid * ROWS_PER_TILE, ROWS_PER_TILE)])
        plsc.subcore_barrier()

        def body(j, carry):
            slot = lax.rem(j, 2)
            row_copy(j, slot).wait()
            idx_copy(j, slot).wait()
            for k in range(SUB):
                scat_copy(k, slot).start(add=True)

            @pl.when(j >= 1)
            def _():
                for k in range(SUB):
                    scat_copy(k, 1 - slot).wait()

            @pl.when(j + 1 < NFULL)
            def _():
                row_copy(j + 1, 1 - slot).start()
                idx_copy(j + 1, 1 - slot).start()

            return carry

        lax.fori_loop(0, NFULL, body, 0)

        def tail_row_copy():
            return pltpu.make_async_copy(
                e_hbm.at[pl.ds(ebase + NFULL * LCHUNK, CHUNK)],
                rows.at[0, pl.ds(0, CHUNK)], lsem.at[0],
            )

        tail_row_copy().start()
        idx_copy(NFULL, 0).start()
        for k in range(SUB):
            scat_copy(k, 1).wait()
        tail_row_copy().wait()
        idx_copy(NFULL, 0).wait()
        scat_copy(0, 0).start(add=True)
        scat_copy(0, 0).wait()
        plsc.subcore_barrier()

        r0 = sid * ROWS_PER_TILE
        pltpu.sync_copy(
            acc.at[pl.ds(r0, ROWS_PER_TILE)], out_hbm.at[cid, pl.ds(r0, ROWS_PER_TILE)]
        )

    return seg(e, dst3, zeros)


BLK = 1000


def _tc_body(p_ref, v_ref, w1_ref, b1_ref, w2_ref, b2_ref, lnw_ref, lnb_ref, out_ref):
    x = p_ref[0] + p_ref[1]
    h = jnp.dot(x, w1_ref[...], preferred_element_type=jnp.float32) + b1_ref[...]
    s = jnp.maximum(h, 0.0) + jnp.log1p(jnp.exp(-jnp.abs(h))) - SHIFT
    y = jnp.dot(s, w2_ref[...], preferred_element_type=jnp.float32) + b2_ref[...]
    mu = jnp.mean(y, axis=-1, keepdims=True)
    yc = y - mu
    var = jnp.mean(yc * yc, axis=-1, keepdims=True)
    out_ref[...] = v_ref[...] + yc * lax.rsqrt(var + 1e-5) * lnw_ref[...] + lnb_ref[...]


def _tc_mlp(partials, v, W1, b1, W2, b2, lnw, lnb):
    return pl.pallas_call(
        _tc_body,
        grid=(N // BLK,),
        in_specs=[
            pl.BlockSpec((NC, BLK, H), lambda i: (0, i, 0)),
            pl.BlockSpec((BLK, H), lambda i: (i, 0)),
            pl.BlockSpec((H, H), lambda i: (0, 0)),
            pl.BlockSpec((1, H), lambda i: (0, 0)),
            pl.BlockSpec((H, H), lambda i: (0, 0)),
            pl.BlockSpec((1, H), lambda i: (0, 0)),
            pl.BlockSpec((1, H), lambda i: (0, 0)),
            pl.BlockSpec((1, H), lambda i: (0, 0)),
        ],
        out_specs=pl.BlockSpec((BLK, H), lambda i: (i, 0)),
        out_shape=jax.ShapeDtypeStruct((N, H), jnp.float32),
    )(partials, v, W1, b1, W2, b2, lnw, lnb)


def kernel(v, e, edge_index, v1_size, W1_1, b1_1, W1_2, b1_2, ln_w, ln_b):
    del v1_size
    dst4 = jnp.pad(
        edge_index[1].reshape(NW, PER_W), ((0, 0), (0, LCHUNK - CHUNK))
    ).reshape(NW, NFULL + 1, SUB, CHUNK)
    zeros = jnp.zeros((ROWS_PER_TILE, H), jnp.float32)
    partials = _sc_segment_sum(e, dst4, zeros)
    return _tc_mlp(
        partials, v, W1_1, b1_1.reshape(1, H), W1_2, b1_2.reshape(1, H),
        ln_w.reshape(1, H), ln_b.reshape(1, H),
    )


# speedup vs baseline: 1.1610x; 1.0142x over previous
import functools

import jax
import jax.numpy as jnp
import numpy as np
from jax import lax
from jax.experimental import pallas as pl
from jax.experimental.pallas import tpu as pltpu
from jax.experimental.pallas import tpu_sc as plsc

N = 10000
E = 320000
H = 128
NF = 128
SHIFT = float(np.log(2.0))

NC = 2
NS = 16
NW = NC * NS
PER_W = E // NW
CHUNK = 80
NCH = PER_W // CHUNK
SUB = 2
LCHUNK = CHUNK * SUB
NFULL = 62
NPAD = 10240
ROWS_PER_TILE = NPAD // NS


def _sc_segment_sum(e, dst3, zeros):
    mesh = plsc.VectorSubcoreMesh(core_axis_name="c", subcore_axis_name="s")

    @functools.partial(
        pl.kernel,
        out_type=jax.ShapeDtypeStruct((NC, NPAD, H), jnp.float32),
        mesh=mesh,
        scratch_types=[
            pltpu.VMEM_SHARED((NPAD, H), jnp.float32),
            pltpu.VMEM((2, LCHUNK, H), jnp.float32),
            pltpu.VMEM((2, SUB, CHUNK), jnp.int32),
            pltpu.SemaphoreType.DMA((2,)),
            pltpu.SemaphoreType.DMA((2,)),
            pltpu.SemaphoreType.DMA((2,)),
        ],
    )
    def seg(e_hbm, dst_hbm, zero_hbm, out_hbm, acc, rows, idx, lsem, isem, ssem):
        cid = lax.axis_index("c")
        sid = lax.axis_index("s")
        w = cid * NS + sid
        ebase = w * PER_W

        def row_copy(j, slot):
            return pltpu.make_async_copy(
                e_hbm.at[pl.ds(ebase + j * LCHUNK, LCHUNK)], rows.at[slot],
                lsem.at[slot],
            )

        def idx_copy(j, slot):
            return pltpu.make_async_copy(
                dst_hbm.at[w, j], idx.at[slot], isem.at[slot]
            )

        def scat_copy(k, slot):
            return pltpu.make_async_copy(
                rows.at[slot, pl.ds(k * CHUNK, CHUNK)],
                acc.at[idx.at[slot, k]], ssem.at[slot],
            )

        row_copy(0, 0).start()
        idx_copy(0, 0).start()
        pltpu.sync_copy(zero_hbm, acc.at[pl.ds(sid * ROWS_PER_TILE, ROWS_PER_TILE)])
        plsc.subcore_barrier()

        def body(j, carry):
            slot = lax.rem(j, 2)
            row_copy(j, slot).wait()
            idx_copy(j, slot).wait()
            for k in range(SUB):
                scat_copy(k, slot).start(add=True)

            @pl.when(j >= 1)
            def _():
                for k in range(SUB):
                    scat_copy(k, 1 - slot).wait()

            @pl.when(j + 1 < NFULL)
            def _():
                row_copy(j + 1, 1 - slot).start()
                idx_copy(j + 1, 1 - slot).start()

            return carry

        lax.fori_loop(0, NFULL, body, 0)

        def tail_row_copy():
            return pltpu.make_async_copy(
                e_hbm.at[pl.ds(ebase + NFULL * LCHUNK, CHUNK)],
                rows.at[0, pl.ds(0, CHUNK)], lsem.at[0],
            )

        tail_row_copy().start()
        idx_copy(NFULL, 0).start()
        for k in range(SUB):
            scat_copy(k, 1).wait()
        tail_row_copy().wait()
        idx_copy(NFULL, 0).wait()
        scat_copy(0, 0).start(add=True)
        scat_copy(0, 0).wait()
        plsc.subcore_barrier()

        r0 = sid * ROWS_PER_TILE
        pltpu.sync_copy(
            acc.at[pl.ds(r0, ROWS_PER_TILE)], out_hbm.at[cid, pl.ds(r0, ROWS_PER_TILE)]
        )

    return seg(e, dst3, zeros)


BLK = 2000


def _tc_body(p_ref, v_ref, w1_ref, b1_ref, w2_ref, b2_ref, lnw_ref, lnb_ref, out_ref):
    x = p_ref[0] + p_ref[1]
    h = jnp.dot(x, w1_ref[...], preferred_element_type=jnp.float32) + b1_ref[...]
    s = jnp.maximum(h, 0.0) + jnp.log1p(jnp.exp(-jnp.abs(h))) - SHIFT
    y = jnp.dot(s, w2_ref[...], preferred_element_type=jnp.float32) + b2_ref[...]
    mu = jnp.mean(y, axis=-1, keepdims=True)
    yc = y - mu
    var = jnp.mean(yc * yc, axis=-1, keepdims=True)
    out_ref[...] = v_ref[...] + yc * lax.rsqrt(var + 1e-5) * lnw_ref[...] + lnb_ref[...]


def _tc_mlp(partials, v, W1, b1, W2, b2, lnw, lnb):
    return pl.pallas_call(
        _tc_body,
        grid=(N // BLK,),
        in_specs=[
            pl.BlockSpec((NC, BLK, H), lambda i: (0, i, 0)),
            pl.BlockSpec((BLK, H), lambda i: (i, 0)),
            pl.BlockSpec((H, H), lambda i: (0, 0)),
            pl.BlockSpec((1, H), lambda i: (0, 0)),
            pl.BlockSpec((H, H), lambda i: (0, 0)),
            pl.BlockSpec((1, H), lambda i: (0, 0)),
            pl.BlockSpec((1, H), lambda i: (0, 0)),
            pl.BlockSpec((1, H), lambda i: (0, 0)),
        ],
        out_specs=pl.BlockSpec((BLK, H), lambda i: (i, 0)),
        out_shape=jax.ShapeDtypeStruct((N, H), jnp.float32),
    )(partials, v, W1, b1, W2, b2, lnw, lnb)


def kernel(v, e, edge_index, v1_size, W1_1, b1_1, W1_2, b1_2, ln_w, ln_b):
    del v1_size
    dst4 = jnp.pad(
        edge_index[1].reshape(NW, PER_W), ((0, 0), (0, LCHUNK - CHUNK))
    ).reshape(NW, NFULL + 1, SUB, CHUNK)
    zeros = jnp.zeros((ROWS_PER_TILE, H), jnp.float32)
    partials = _sc_segment_sum(e, dst4, zeros)
    return _tc_mlp(
        partials, v, W1_1, b1_1.reshape(1, H), W1_2, b1_2.reshape(1, H),
        ln_w.reshape(1, H), ln_b.reshape(1, H),
    )


# speedup vs baseline: 1.1727x; 1.0101x over previous
import functools

import jax
import jax.numpy as jnp
import numpy as np
from jax import lax
from jax.experimental import pallas as pl
from jax.experimental.pallas import tpu as pltpu
from jax.experimental.pallas import tpu_sc as plsc

N = 10000
E = 320000
H = 128
NF = 128
SHIFT = float(np.log(2.0))

NC = 2
NS = 16
NW = NC * NS
PER_W = E // NW
CHUNK = 80
NCH = PER_W // CHUNK
SUB = 2
LCHUNK = CHUNK * SUB
NFULL = 62
NPAD = 10240
ROWS_PER_TILE = NPAD // NS


def _sc_segment_sum(e, dst3, zeros):
    mesh = plsc.VectorSubcoreMesh(core_axis_name="c", subcore_axis_name="s")

    @functools.partial(
        pl.kernel,
        out_type=jax.ShapeDtypeStruct((NC, NPAD, H), jnp.float32),
        mesh=mesh,
        compiler_params=pltpu.CompilerParams(use_tc_tiling_on_sc=False),
        scratch_types=[
            pltpu.VMEM_SHARED((NPAD, H), jnp.float32),
            pltpu.VMEM((2, LCHUNK, H), jnp.float32),
            pltpu.VMEM((2, SUB, CHUNK), jnp.int32),
            pltpu.SemaphoreType.DMA((2,)),
            pltpu.SemaphoreType.DMA((2,)),
            pltpu.SemaphoreType.DMA((2,)),
        ],
    )
    def seg(e_hbm, dst_hbm, zero_hbm, out_hbm, acc, rows, idx, lsem, isem, ssem):
        cid = lax.axis_index("c")
        sid = lax.axis_index("s")
        w = cid * NS + sid
        ebase = w * PER_W

        def row_copy(j, slot):
            return pltpu.make_async_copy(
                e_hbm.at[pl.ds(ebase + j * LCHUNK, LCHUNK)], rows.at[slot],
                lsem.at[slot],
            )

        def idx_copy(j, slot):
            return pltpu.make_async_copy(
                dst_hbm.at[w, j], idx.at[slot], isem.at[slot]
            )

        def scat_copy(k, slot):
            return pltpu.make_async_copy(
                rows.at[slot, pl.ds(k * CHUNK, CHUNK)],
                acc.at[idx.at[slot, k]], ssem.at[slot],
            )

        row_copy(0, 0).start()
        idx_copy(0, 0).start()
        pltpu.sync_copy(zero_hbm, acc.at[pl.ds(sid * ROWS_PER_TILE, ROWS_PER_TILE)])
        plsc.subcore_barrier()

        def body(j, carry):
            slot = lax.rem(j, 2)
            row_copy(j, slot).wait()
            idx_copy(j, slot).wait()
            for k in range(SUB):
                scat_copy(k, slot).start(add=True)

            @pl.when(j >= 1)
            def _():
                for k in range(SUB):
                    scat_copy(k, 1 - slot).wait()

            @pl.when(j + 1 < NFULL)
            def _():
                row_copy(j + 1, 1 - slot).start()
                idx_copy(j + 1, 1 - slot).start()

            return carry

        lax.fori_loop(0, NFULL, body, 0)

        def tail_row_copy():
            return pltpu.make_async_copy(
                e_hbm.at[pl.ds(ebase + NFULL * LCHUNK, CHUNK)],
                rows.at[0, pl.ds(0, CHUNK)], lsem.at[0],
            )

        tail_row_copy().start()
        idx_copy(NFULL, 0).start()
        for k in range(SUB):
            scat_copy(k, 1).wait()
        tail_row_copy().wait()
        idx_copy(NFULL, 0).wait()
        scat_copy(0, 0).start(add=True)
        scat_copy(0, 0).wait()
        plsc.subcore_barrier()

        r0 = sid * ROWS_PER_TILE
        pltpu.sync_copy(
            acc.at[pl.ds(r0, ROWS_PER_TILE)], out_hbm.at[cid, pl.ds(r0, ROWS_PER_TILE)]
        )

    return seg(e, dst3, zeros)


BLK = 2000


def _tc_body(p_ref, v_ref, w1_ref, b1_ref, w2_ref, b2_ref, lnw_ref, lnb_ref, out_ref):
    x = p_ref[0] + p_ref[1]
    h = jnp.dot(x, w1_ref[...], preferred_element_type=jnp.float32) + b1_ref[...]
    s = jnp.maximum(h, 0.0) + jnp.log1p(jnp.exp(-jnp.abs(h))) - SHIFT
    y = jnp.dot(s, w2_ref[...], preferred_element_type=jnp.float32) + b2_ref[...]
    mu = jnp.mean(y, axis=-1, keepdims=True)
    yc = y - mu
    var = jnp.mean(yc * yc, axis=-1, keepdims=True)
    out_ref[...] = v_ref[...] + yc * lax.rsqrt(var + 1e-5) * lnw_ref[...] + lnb_ref[...]


def _tc_mlp(partials, v, W1, b1, W2, b2, lnw, lnb):
    return pl.pallas_call(
        _tc_body,
        grid=(N // BLK,),
        in_specs=[
            pl.BlockSpec((NC, BLK, H), lambda i: (0, i, 0)),
            pl.BlockSpec((BLK, H), lambda i: (i, 0)),
            pl.BlockSpec((H, H), lambda i: (0, 0)),
            pl.BlockSpec((1, H), lambda i: (0, 0)),
            pl.BlockSpec((H, H), lambda i: (0, 0)),
            pl.BlockSpec((1, H), lambda i: (0, 0)),
            pl.BlockSpec((1, H), lambda i: (0, 0)),
            pl.BlockSpec((1, H), lambda i: (0, 0)),
        ],
        out_specs=pl.BlockSpec((BLK, H), lambda i: (i, 0)),
        out_shape=jax.ShapeDtypeStruct((N, H), jnp.float32),
    )(partials, v, W1, b1, W2, b2, lnw, lnb)


def kernel(v, e, edge_index, v1_size, W1_1, b1_1, W1_2, b1_2, ln_w, ln_b):
    del v1_size
    dst4 = jnp.pad(
        edge_index[1].reshape(NW, PER_W), ((0, 0), (0, LCHUNK - CHUNK))
    ).reshape(NW, NFULL + 1, SUB, CHUNK)
    zeros = jnp.zeros((ROWS_PER_TILE, H), jnp.float32)
    partials = _sc_segment_sum(e, dst4, zeros)
    return _tc_mlp(
        partials, v, W1_1, b1_1.reshape(1, H), W1_2, b1_2.reshape(1, H),
        ln_w.reshape(1, H), ln_b.reshape(1, H),
    )


# speedup vs baseline: 1.2701x; 1.0830x over previous
import functools

import jax
import jax.numpy as jnp
import numpy as np
from jax import lax
from jax.experimental import pallas as pl
from jax.experimental.pallas import tpu as pltpu
from jax.experimental.pallas import tpu_sc as plsc

N = 10000
E = 320000
H = 128
NF = 128
SHIFT = float(np.log(2.0))

NC = 2
NS = 16
NW = NC * NS
PER_W = E // NW
CHUNK = 80
NCH = PER_W // CHUNK
SUB = 2
LCHUNK = CHUNK * SUB
NFULL = 62
NPAD = 10240
ROWS_PER_TILE = NPAD // NS


def _sc_segment_sum(e, dst3, zeros):
    mesh = plsc.VectorSubcoreMesh(core_axis_name="c", subcore_axis_name="s")

    @functools.partial(
        pl.kernel,
        out_type=jax.ShapeDtypeStruct((NC, NPAD, H), jnp.float32),
        mesh=mesh,
        compiler_params=pltpu.CompilerParams(use_tc_tiling_on_sc=False),
        scratch_types=[
            pltpu.VMEM_SHARED((NPAD, H), jnp.float32),
            pltpu.VMEM((2, LCHUNK, H), jnp.float32),
            pltpu.VMEM((2, SUB, CHUNK), jnp.int32),
            pltpu.SemaphoreType.DMA((2,)),
            pltpu.SemaphoreType.DMA((2,)),
            pltpu.SemaphoreType.DMA((2,)),
        ],
    )
    def seg(e_hbm, dst_hbm, zero_hbm, out_hbm, acc, rows, idx, lsem, isem, ssem):
        cid = lax.axis_index("c")
        sid = lax.axis_index("s")
        w = cid * NS + sid
        ebase = w * PER_W

        def row_copy(j, slot):
            return pltpu.make_async_copy(
                e_hbm.at[pl.ds(ebase + j * LCHUNK, LCHUNK)], rows.at[slot],
                lsem.at[slot],
            )

        def idx_copy(j, slot):
            return pltpu.make_async_copy(
                dst_hbm.at[1, w, pl.ds(j * SUB, SUB)], idx.at[slot], isem.at[slot]
            )

        def tail_idx_copy():
            return pltpu.make_async_copy(
                dst_hbm.at[1, w, pl.ds(NFULL * SUB, 1)], idx.at[0, pl.ds(0, 1)],
                isem.at[0],
            )

        def scat_copy(k, slot):
            return pltpu.make_async_copy(
                rows.at[slot, pl.ds(k * CHUNK, CHUNK)],
                acc.at[idx.at[slot, k]], ssem.at[slot],
            )

        row_copy(0, 0).start()
        idx_copy(0, 0).start()
        pltpu.sync_copy(zero_hbm, acc.at[pl.ds(sid * ROWS_PER_TILE, ROWS_PER_TILE)])
        plsc.subcore_barrier()

        def body(j, carry):
            slot = lax.rem(j, 2)
            row_copy(j, slot).wait()
            idx_copy(j, slot).wait()
            for k in range(SUB):
                scat_copy(k, slot).start(add=True)

            @pl.when(j >= 1)
            def _():
                for k in range(SUB):
                    scat_copy(k, 1 - slot).wait()

            @pl.when(j + 1 < NFULL)
            def _():
                row_copy(j + 1, 1 - slot).start()
                idx_copy(j + 1, 1 - slot).start()

            return carry

        lax.fori_loop(0, NFULL, body, 0)

        def tail_row_copy():
            return pltpu.make_async_copy(
                e_hbm.at[pl.ds(ebase + NFULL * LCHUNK, CHUNK)],
                rows.at[0, pl.ds(0, CHUNK)], lsem.at[0],
            )

        tail_row_copy().start()
        tail_idx_copy().start()
        for k in range(SUB):
            scat_copy(k, 1).wait()
        tail_row_copy().wait()
        tail_idx_copy().wait()
        scat_copy(0, 0).start(add=True)
        scat_copy(0, 0).wait()
        plsc.subcore_barrier()

        r0 = sid * ROWS_PER_TILE
        pltpu.sync_copy(
            acc.at[pl.ds(r0, ROWS_PER_TILE)], out_hbm.at[cid, pl.ds(r0, ROWS_PER_TILE)]
        )

    return seg(e, dst3, zeros)


BLK = 2000


def _tc_body(p_ref, v_ref, w1_ref, b1_ref, w2_ref, b2_ref, lnw_ref, lnb_ref, out_ref):
    x = p_ref[0] + p_ref[1]
    h = jnp.dot(x, w1_ref[...], preferred_element_type=jnp.float32) + b1_ref[...]
    s = jnp.maximum(h, 0.0) + jnp.log1p(jnp.exp(-jnp.abs(h))) - SHIFT
    y = jnp.dot(s, w2_ref[...], preferred_element_type=jnp.float32) + b2_ref[...]
    mu = jnp.mean(y, axis=-1, keepdims=True)
    yc = y - mu
    var = jnp.mean(yc * yc, axis=-1, keepdims=True)
    out_ref[...] = v_ref[...] + yc * lax.rsqrt(var + 1e-5) * lnw_ref[...] + lnb_ref[...]


def _tc_mlp(partials, v, W1, b1, W2, b2, lnw, lnb):
    return pl.pallas_call(
        _tc_body,
        grid=(N // BLK,),
        in_specs=[
            pl.BlockSpec((NC, BLK, H), lambda i: (0, i, 0)),
            pl.BlockSpec((BLK, H), lambda i: (i, 0)),
            pl.BlockSpec((H, H), lambda i: (0, 0)),
            pl.BlockSpec((1, H), lambda i: (0, 0)),
            pl.BlockSpec((H, H), lambda i: (0, 0)),
            pl.BlockSpec((1, H), lambda i: (0, 0)),
            pl.BlockSpec((1, H), lambda i: (0, 0)),
            pl.BlockSpec((1, H), lambda i: (0, 0)),
        ],
        out_specs=pl.BlockSpec((BLK, H), lambda i: (i, 0)),
        out_shape=jax.ShapeDtypeStruct((N, H), jnp.float32),
    )(partials, v, W1, b1, W2, b2, lnw, lnb)


def kernel(v, e, edge_index, v1_size, W1_1, b1_1, W1_2, b1_2, ln_w, ln_b):
    del v1_size
    dst = edge_index.reshape(2, NW, NCH, CHUNK)
    zeros = jnp.zeros((ROWS_PER_TILE, H), jnp.float32)
    partials = _sc_segment_sum(e, dst, zeros)
    return _tc_mlp(
        partials, v, W1_1, b1_1.reshape(1, H), W1_2, b1_2.reshape(1, H),
        ln_w.reshape(1, H), ln_b.reshape(1, H),
    )
